# baseline (speedup 1.0000x reference)
import functools

import jax
import jax.numpy as jnp
from jax.experimental import pallas as pl
from jax.experimental.pallas import tpu as pltpu

_CHUNK = 128


def _body(x0_ref, x1t_ref, out_ref, colacc_ref, s_ref, *, b0, n1, g0, n0):
    i = pl.program_id(0)

    x0b = x0_ref[...]
    sq0b = x0_ref[:, 3:4]

    r_part = None
    c_parts = []
    for k in range(n1 // _CHUNK):
        x1c = x1t_ref[:, k * _CHUNK:(k + 1) * _CHUNK]
        dk = jnp.dot(x0b, x1c, preferred_element_type=jnp.float32)
        sq1c = x1t_ref[7:8, k * _CHUNK:(k + 1) * _CHUNK]
        d2 = (dk + sq0b) + sq1c

        t = d2
        w = _CHUNK
        while w > 128:
            w //= 2
            t = jnp.minimum(t[:, :w], t[:, w:])
        r_part = t if r_part is None else jnp.minimum(r_part, t)

        c = d2
        h = b0
        while h > 8:
            h //= 2
            c = jnp.minimum(c[:h, :], c[h:, :])
        c_parts.append(c)

    r_min = jnp.min(r_part, axis=1, keepdims=True)
    s_i = jnp.sum(jnp.maximum(r_min, 0.0))

    @pl.when(i == 0)
    def _():
        s_ref[0] = s_i

    @pl.when(i > 0)
    def _():
        s_ref[0] = s_ref[0] + s_i

    c_part = jnp.concatenate(c_parts, axis=1)

    @pl.when(i == 0)
    def _():
        colacc_ref[...] = c_part

    @pl.when(i > 0)
    def _():
        colacc_ref[...] = jnp.minimum(colacc_ref[...], c_part)

    @pl.when(i == g0 - 1)
    def _():
        c = colacc_ref[...]
        c = jnp.minimum(c[:4, :], c[4:, :])
        c = jnp.minimum(c[:2, :], c[2:, :])
        c = jnp.minimum(c[:1, :], c[1:, :])
        s1 = jnp.sum(jnp.maximum(c, 0.0)) / n1
        out_ref[...] = (s_ref[0] / n0 + s1).reshape(1, 1)


@jax.jit
def kernel(input0, input1):
    n0 = input0.shape[0]
    n1 = input1.shape[0]
    b0 = 2048
    g0 = n0 // b0

    f32 = jnp.float32
    sq0 = jnp.sum(input0 * input0, axis=1, keepdims=True)
    sq1 = jnp.sum(input1 * input1, axis=1, keepdims=True)
    x0 = jnp.concatenate(
        [input0, sq0, jnp.zeros((n0, 4), f32)], axis=1)
    x1t = jnp.concatenate(
        [-2.0 * input1, jnp.zeros((n1, 4), f32), sq1], axis=1).T

    body = functools.partial(_body, b0=b0, n1=n1, g0=g0, n0=n0)
    out = pl.pallas_call(
        body,
        grid=(g0,),
        in_specs=[
            pl.BlockSpec((b0, 8), lambda i: (i, 0)),
            pl.BlockSpec((8, n1), lambda i: (0, 0)),
        ],
        out_specs=pl.BlockSpec((1, 1), lambda i: (0, 0)),
        out_shape=jax.ShapeDtypeStruct((1, 1), f32),
        scratch_shapes=[
            pltpu.VMEM((8, n1), f32),
            pltpu.SMEM((1,), f32),
        ],
    )(x0, x1t)
    return out[0, 0]


# speedup vs baseline: 1.1261x; 1.1261x over previous
import functools

import jax
import jax.numpy as jnp
from jax.experimental import pallas as pl
from jax.experimental.pallas import tpu as pltpu

_CHUNK = 256


def _body(x0_ref, x1t_ref, out_ref, colacc_ref, s_ref, *, b0, n1, g0, n0):
    i = pl.program_id(0)

    x0b = x0_ref[...]
    sq0b = x0_ref[:, 3:4]

    r_part = None
    c_parts = []
    for k in range(n1 // _CHUNK):
        x1c = x1t_ref[:, k * _CHUNK:(k + 1) * _CHUNK]
        dk = jnp.dot(x0b, x1c, preferred_element_type=jnp.float32)
        sq1c = x1t_ref[7:8, k * _CHUNK:(k + 1) * _CHUNK]
        d2 = (dk + sq0b) + sq1c

        t = d2
        w = _CHUNK
        while w > 128:
            w //= 2
            t = jnp.minimum(t[:, :w], t[:, w:])
        r_part = t if r_part is None else jnp.minimum(r_part, t)

        c = d2
        h = b0
        while h > 8:
            h //= 2
            c = jnp.minimum(c[:h, :], c[h:, :])
        c_parts.append(c)

    r_min = jnp.min(r_part, axis=1, keepdims=True)
    s_i = jnp.sum(jnp.maximum(r_min, 0.0))

    @pl.when(i == 0)
    def _():
        s_ref[0] = s_i

    @pl.when(i > 0)
    def _():
        s_ref[0] = s_ref[0] + s_i

    c_part = jnp.concatenate(c_parts, axis=1)

    @pl.when(i == 0)
    def _():
        colacc_ref[...] = c_part

    @pl.when(i > 0)
    def _():
        colacc_ref[...] = jnp.minimum(colacc_ref[...], c_part)

    @pl.when(i == g0 - 1)
    def _():
        c = colacc_ref[...]
        c = jnp.minimum(c[:4, :], c[4:, :])
        c = jnp.minimum(c[:2, :], c[2:, :])
        c = jnp.minimum(c[:1, :], c[1:, :])
        s1 = jnp.sum(jnp.maximum(c, 0.0)) / n1
        out_ref[...] = (s_ref[0] / n0 + s1).reshape(1, 1)


@jax.jit
def kernel(input0, input1):
    n0 = input0.shape[0]
    n1 = input1.shape[0]
    b0 = 4096
    g0 = n0 // b0

    f32 = jnp.float32
    sq0 = jnp.sum(input0 * input0, axis=1, keepdims=True)
    sq1 = jnp.sum(input1 * input1, axis=1, keepdims=True)
    x0 = jnp.concatenate(
        [input0, sq0, jnp.zeros((n0, 4), f32)], axis=1)
    x1t = jnp.concatenate(
        [-2.0 * input1, jnp.zeros((n1, 4), f32), sq1], axis=1).T

    body = functools.partial(_body, b0=b0, n1=n1, g0=g0, n0=n0)
    out = pl.pallas_call(
        body,
        grid=(g0,),
        in_specs=[
            pl.BlockSpec((b0, 8), lambda i: (i, 0)),
            pl.BlockSpec((8, n1), lambda i: (0, 0)),
        ],
        out_specs=pl.BlockSpec((1, 1), lambda i: (0, 0)),
        out_shape=jax.ShapeDtypeStruct((1, 1), f32),
        scratch_shapes=[
            pltpu.VMEM((8, n1), f32),
            pltpu.SMEM((1,), f32),
        ],
    )(x0, x1t)
    return out[0, 0]
